# baseline (speedup 1.0000x reference)
import functools

import jax
import jax.numpy as jnp
from jax import lax
from jax.experimental import pallas as pl
from jax.experimental.pallas import tpu as pltpu
from jax.experimental.pallas import tpu_sc as plsc


def kernel(input_words, output_words, in_embed, out_embed):
    B, = input_words.shape
    V, D = in_embed.shape

    info = plsc.get_sparse_core_info()
    nc, ns = info.num_cores, info.num_subcores
    nw = nc * ns
    bpw = B // nw

    mesh = plsc.VectorSubcoreMesh(core_axis_name="c", subcore_axis_name="s")

    @functools.partial(
        pl.kernel,
        mesh=mesh,
        compiler_params=pltpu.CompilerParams(use_tc_tiling_on_sc=False),
        out_type=jax.ShapeDtypeStruct((2, B, D), jnp.float32),
        scratch_types=[
            pltpu.VMEM((bpw,), jnp.int32),
            pltpu.VMEM((bpw,), jnp.int32),
            pltpu.VMEM((bpw, D), jnp.float32),
            pltpu.VMEM((bpw, D), jnp.float32),
            pltpu.SemaphoreType.DMA,
            pltpu.SemaphoreType.DMA,
        ],
    )
    def _gather2(iw_hbm, ow_hbm, ie_hbm, oe_hbm, out_hbm,
                 idx_in, idx_out, rows_in, rows_out, sem_in, sem_out):
        wid = lax.axis_index("s") * nc + lax.axis_index("c")
        base = wid * bpw
        pltpu.sync_copy(iw_hbm.at[pl.ds(base, bpw)], idx_in)
        pltpu.sync_copy(ow_hbm.at[pl.ds(base, bpw)], idx_out)
        cp_in = pltpu.async_copy(ie_hbm.at[idx_in], rows_in, sem_in)
        cp_out = pltpu.async_copy(oe_hbm.at[idx_out], rows_out, sem_out)
        cp_in.wait()
        pltpu.sync_copy(rows_in, out_hbm.at[0, pl.ds(base, bpw)])
        cp_out.wait()
        pltpu.sync_copy(rows_out, out_hbm.at[1, pl.ds(base, bpw)])

    return _gather2(input_words, output_words, in_embed, out_embed)


# speedup vs baseline: 1.5832x; 1.5832x over previous
import functools

import jax
import jax.numpy as jnp
from jax import lax
from jax.experimental import pallas as pl
from jax.experimental.pallas import tpu as pltpu
from jax.experimental.pallas import tpu_sc as plsc


def kernel(input_words, output_words, in_embed, out_embed):
    B, = input_words.shape
    V, D = in_embed.shape

    info = plsc.get_sparse_core_info()
    nc, ns, L = info.num_cores, info.num_subcores, info.num_lanes
    nw = nc * ns
    bpw = B // nw

    mesh = plsc.VectorSubcoreMesh(core_axis_name="c", subcore_axis_name="s")

    @functools.partial(
        pl.kernel,
        mesh=mesh,
        out_type=jax.ShapeDtypeStruct((2, B, D), jnp.float32),
        scratch_types=[
            pltpu.VMEM((bpw,), jnp.int32),
            pltpu.VMEM((bpw,), jnp.int32),
            pltpu.VMEM((bpw, D), jnp.float32),
            pltpu.SemaphoreType.DMA,
        ],
    )
    def _gather2(iw_hbm, ow_hbm, ie_hbm, oe_hbm, out_hbm,
                 idx0, idx1, buf, sem):
        wid = lax.axis_index("s") * nc + lax.axis_index("c")
        base = wid * bpw
        pltpu.sync_copy(iw_hbm.at[pl.ds(base, bpw)], idx0)
        pltpu.sync_copy(ow_hbm.at[pl.ds(base, bpw)], idx1)

        for table, idx, out_row in ((ie_hbm, idx0, 0), (oe_hbm, idx1, 1)):
            def issue(k, _, table=table, idx=idx):
                v = idx[pl.ds(k * L, L)]
                for j in range(L):
                    pltpu.async_copy(
                        table.at[pl.ds(v[j], 1)],
                        buf.at[pl.ds(k * L + j, 1)],
                        sem)
                return 0
            lax.fori_loop(0, bpw // L, issue, 0)
            pltpu.make_async_copy(ie_hbm.at[pl.ds(0, bpw)], buf, sem).wait()
            pltpu.sync_copy(buf, out_hbm.at[out_row, pl.ds(base, bpw)])

    return _gather2(input_words, output_words, in_embed, out_embed)
